# Initial kernel scaffold; baseline (speedup 1.0000x reference)
import jax
import jax.numpy as jnp
from jax.experimental import pallas as pl


def kernel(x, edge_index, edge_weight, W, b):
    raise NotImplementedError("write your pallas kernel here")


# speedup vs baseline: 2.5497x; 2.5497x over previous
import functools
import jax
import jax.numpy as jnp
from jax import lax
from jax.experimental import pallas as pl
from jax.experimental.pallas import tpu as pltpu
from jax.experimental.pallas import tpu_sc as plsc

N = 10000
NP = 10240
E = 320000
F = 128
H = 64
L = 3
NS = 16
NC = 2
RPT = NP // NS
EPT = E // NS
IW = 100
JR = 8
CH = JR * IW
NCHUNK = EPT // CH
ZR = 80

_mesh = plsc.VectorSubcoreMesh(core_axis_name="c", subcore_axis_name="s")


@functools.partial(
    pl.kernel,
    out_type=jax.ShapeDtypeStruct((L, NC * NP, H), jnp.float32),
    mesh=_mesh,
    compiler_params=pltpu.CompilerParams(use_tc_tiling_on_sc=False),
    scratch_types=[
        pltpu.VMEM_SHARED((NP, H), jnp.float32),
        pltpu.VMEM((JR, IW), jnp.int32),
        pltpu.VMEM((JR, IW), jnp.int32),
        pltpu.VMEM((CH,), jnp.float32),
        pltpu.VMEM((CH, H), jnp.float32),
        pltpu.VMEM((ZR, H), jnp.float32),
    ],
)
def _gcn_sc(xs, src2, dst2, wf, out, acc_sp, src_vm, dst_vm, w_vm, rows_vm,
            zeros_vm):
    c = lax.axis_index("c")
    s = lax.axis_index("s")
    row0 = s * RPT

    zv = jnp.zeros((16,), jnp.float32)

    def zbody(r, carry):
        for k in range(H // 16):
            zeros_vm[r, pl.ds(k * 16, 16)] = zv
        return carry

    lax.fori_loop(0, ZR, zbody, 0)

    def zero_acc():
        for z in range(RPT // ZR):
            pltpu.sync_copy(zeros_vm, acc_sp.at[pl.ds(row0 + z * ZR, ZR)])

    zero_acc()
    plsc.subcore_barrier()

    def do_layer(h_ref, lyr):
        def chunk_body(cidx, carry):
            e0 = s * EPT + cidx * CH
            i0 = s * (EPT // IW) + cidx * JR
            pltpu.sync_copy(src2.at[c].at[pl.ds(i0, JR)], src_vm)
            pltpu.sync_copy(dst2.at[pl.ds(i0, JR)], dst_vm)
            pltpu.sync_copy(wf.at[pl.ds(e0, CH)], w_vm)
            for j in range(JR):
                pltpu.sync_copy(h_ref.at[src_vm.at[j]],
                                rows_vm.at[pl.ds(j * IW, IW)])

            def scale(g, carry2):
                wv = w_vm[pl.ds(g * 16, 16)]
                for j in range(16):
                    w = wv[j]
                    e = g * 16 + j
                    for k in range(H // 16):
                        sl = pl.ds(k * 16, 16)
                        rows_vm[e, sl] = rows_vm[e, sl] * w
                return carry2

            lax.fori_loop(0, CH // 16, scale, 0)
            for j in range(JR):
                pltpu.sync_copy(rows_vm.at[pl.ds(j * IW, IW)],
                                acc_sp.at[dst_vm.at[j]], add=True)
            return carry

        lax.fori_loop(0, NCHUNK, chunk_body, 0)
        plsc.subcore_barrier()

        pltpu.sync_copy(acc_sp.at[pl.ds(row0, RPT)], rows_vm.at[pl.ds(0, RPT)])
        pltpu.sync_copy(rows_vm.at[pl.ds(0, RPT)],
                        out.at[lyr].at[pl.ds(c * NP + row0, RPT)])
        if lyr < L - 1:
            zero_acc()
        plsc.subcore_barrier()

    do_layer(xs, 0)
    do_layer(out.at[0], 1)
    do_layer(out.at[1], 2)


RB = 640


def _mm_body(hs_ref, wt_ref, b_ref, o_ref):
    hs = hs_ref[...]
    s0 = hs[0, 0] + hs[1, 0] + hs[2, 0]
    s1 = hs[0, 1] + hs[1, 1] + hs[2, 1]
    m = jnp.concatenate([s0, s1], axis=-1) * (1.0 / 3.0)
    o_ref[...] = jnp.dot(m, wt_ref[...],
                         preferred_element_type=jnp.float32) + b_ref[...]


def _mean_linear(hs4, Wt, b2):
    return pl.pallas_call(
        _mm_body,
        grid=(NP // RB,),
        in_specs=[
            pl.BlockSpec((L, NC, RB, H), lambda i: (0, 0, i, 0)),
            pl.BlockSpec((F, F), lambda i: (0, 0)),
            pl.BlockSpec((1, F), lambda i: (0, 0)),
        ],
        out_specs=pl.BlockSpec((RB, F), lambda i: (i, 0)),
        out_shape=jax.ShapeDtypeStruct((NP, F), jnp.float32),
    )(hs4, Wt, b2)


@jax.jit
def kernel(x, edge_index, edge_weight, W, b):
    xs = x.reshape(N, NC, H).transpose(1, 0, 2)
    xs = jnp.pad(xs, ((0, 0), (0, NP - N), (0, 0))).reshape(NC * NP, H)
    src = edge_index[1]
    src2 = (src[None, :] + jnp.array([[0], [NP]], jnp.int32)).reshape(
        NC, E // IW, IW)
    dst2 = edge_index[0].reshape(E // IW, IW)
    hs = _gcn_sc(xs, src2, dst2, edge_weight)
    hs4 = hs.reshape(L, NC, NP, H)
    return _mean_linear(hs4, W.T, b.reshape(1, F))[:N]


# speedup vs baseline: 4.3420x; 1.7030x over previous
import functools
import jax
import jax.numpy as jnp
from jax import lax
from jax.experimental import pallas as pl
from jax.experimental.pallas import tpu as pltpu
from jax.experimental.pallas import tpu_sc as plsc

N = 10000
NP = 10240
E = 320000
F = 128
H = 64
L = 3
NS = 16
NC = 2
RPT = NP // NS
EPT = E // NS
IW = 100
JR = 4
CH = JR * IW
PAIR = 2 * CH
NPAIR = EPT // PAIR
ZR = 80

_mesh = plsc.VectorSubcoreMesh(core_axis_name="c", subcore_axis_name="s")


@functools.partial(
    pl.kernel,
    out_type=jax.ShapeDtypeStruct((L, NC * NP, H), jnp.float32),
    mesh=_mesh,
    compiler_params=pltpu.CompilerParams(use_tc_tiling_on_sc=False),
    scratch_types=[
        pltpu.VMEM_SHARED((NP, H), jnp.float32),
        pltpu.VMEM((2 * JR, IW), jnp.int32),
        pltpu.VMEM((2 * JR, IW), jnp.int32),
        pltpu.VMEM((PAIR,), jnp.float32),
        pltpu.VMEM((2, CH, H), jnp.float32),
        pltpu.VMEM((ZR, H), jnp.float32),
        pltpu.SemaphoreType.DMA,
        pltpu.SemaphoreType.DMA,
        pltpu.SemaphoreType.DMA,
        pltpu.SemaphoreType.DMA,
    ],
)
def _gcn_sc(xs, src2, dst2, wf, out, acc_sp, src_vm, dst_vm, w_vm, rows_vm,
            zeros_vm, gsem0, gsem1, ssem0, ssem1):
    c = lax.axis_index("c")
    s = lax.axis_index("s")
    row0 = s * RPT
    gsems = (gsem0, gsem1)
    ssems = (ssem0, ssem1)

    zv = jnp.zeros((16,), jnp.float32)

    def zbody(r, carry):
        for k in range(H // 16):
            zeros_vm[r, pl.ds(k * 16, 16)] = zv
        return carry

    lax.fori_loop(0, ZR, zbody, 0)

    def zero_acc():
        for z in range(RPT // ZR):
            pltpu.sync_copy(zeros_vm, acc_sp.at[pl.ds(row0 + z * ZR, ZR)])

    zero_acc()
    plsc.subcore_barrier()

    def do_layer(h_ref, lyr):
        def scale(b):
            def s16(g, carry2):
                wv = w_vm[pl.ds(b * CH + g * 16, 16)]
                for j in range(16):
                    w = wv[j]
                    e = g * 16 + j
                    for k in range(H // 16):
                        sl = pl.ds(k * 16, 16)
                        rows_vm[b, e, sl] = rows_vm[b, e, sl] * w
                return carry2

            lax.fori_loop(0, CH // 16, s16, 0)

        def pair_body(p, carry):
            e0 = s * EPT + p * PAIR
            i0 = s * (EPT // IW) + p * (2 * JR)
            pltpu.sync_copy(src2.at[c].at[pl.ds(i0, 2 * JR)], src_vm)
            pltpu.sync_copy(dst2.at[pl.ds(i0, 2 * JR)], dst_vm)
            pltpu.sync_copy(wf.at[pl.ds(e0, PAIR)], w_vm)
            gds = [[], []]
            for b in range(2):
                for j in range(JR):
                    gds[b].append(pltpu.async_copy(
                        h_ref.at[src_vm.at[b * JR + j]],
                        rows_vm.at[b].at[pl.ds(j * IW, IW)], gsems[b]))
            sds = [[], []]
            for b in range(2):
                for d in gds[b]:
                    d.wait()
                scale(b)
                for j in range(JR):
                    sds[b].append(pltpu.async_copy(
                        rows_vm.at[b].at[pl.ds(j * IW, IW)],
                        acc_sp.at[dst_vm.at[b * JR + j]], ssems[b],
                        add=True))
            for b in range(2):
                for d in sds[b]:
                    d.wait()
            return carry

        lax.fori_loop(0, NPAIR, pair_body, 0)
        plsc.subcore_barrier()

        pltpu.sync_copy(acc_sp.at[pl.ds(row0, RPT)],
                        out.at[lyr].at[pl.ds(c * NP + row0, RPT)])
        if lyr < L - 1:
            zero_acc()
        plsc.subcore_barrier()

    do_layer(xs, 0)
    do_layer(out.at[0], 1)
    do_layer(out.at[1], 2)


RB = 640


def _mm_body(hs_ref, wt_ref, b_ref, o_ref):
    hs = hs_ref[...]
    s0 = hs[0, 0] + hs[1, 0] + hs[2, 0]
    s1 = hs[0, 1] + hs[1, 1] + hs[2, 1]
    m = jnp.concatenate([s0, s1], axis=-1) * (1.0 / 3.0)
    o_ref[...] = jnp.dot(m, wt_ref[...],
                         preferred_element_type=jnp.float32) + b_ref[...]


def _mean_linear(hs4, Wt, b2):
    return pl.pallas_call(
        _mm_body,
        grid=(NP // RB,),
        in_specs=[
            pl.BlockSpec((L, NC, RB, H), lambda i: (0, 0, i, 0)),
            pl.BlockSpec((F, F), lambda i: (0, 0)),
            pl.BlockSpec((1, F), lambda i: (0, 0)),
        ],
        out_specs=pl.BlockSpec((RB, F), lambda i: (i, 0)),
        out_shape=jax.ShapeDtypeStruct((NP, F), jnp.float32),
    )(hs4, Wt, b2)


@jax.jit
def kernel(x, edge_index, edge_weight, W, b):
    xs = x.reshape(N, NC, H).transpose(1, 0, 2)
    xs = jnp.pad(xs, ((0, 0), (0, NP - N), (0, 0))).reshape(NC * NP, H)
    src = edge_index[1]
    src2 = (src[None, :] + jnp.array([[0], [NP]], jnp.int32)).reshape(
        NC, E // IW, IW)
    dst2 = edge_index[0].reshape(E // IW, IW)
    hs = _gcn_sc(xs, src2, dst2, edge_weight)
    hs4 = hs.reshape(L, NC, NP, H)
    return _mean_linear(hs4, W.T, b.reshape(1, F))[:N]


# speedup vs baseline: 4.9368x; 1.1370x over previous
import functools
import jax
import jax.numpy as jnp
from jax import lax
from jax.experimental import pallas as pl
from jax.experimental.pallas import tpu as pltpu
from jax.experimental.pallas import tpu_sc as plsc

N = 10000
NP = 10240
E = 320000
F = 128
H = 64
L = 3
NS = 16
NC = 2
RPT = NP // NS
EPT = E // NS
IW = 100
JR = 4
CH = JR * IW
PAIR = 2 * CH
NPAIR = EPT // PAIR
ZR = 80

_mesh = plsc.VectorSubcoreMesh(core_axis_name="c", subcore_axis_name="s")


@functools.partial(
    pl.kernel,
    out_type=jax.ShapeDtypeStruct((L, NC * NP, H), jnp.float32),
    mesh=_mesh,
    compiler_params=pltpu.CompilerParams(use_tc_tiling_on_sc=False),
    scratch_types=[
        pltpu.VMEM_SHARED((NP, H), jnp.float32),
        pltpu.VMEM((2, 2 * JR, IW), jnp.int32),
        pltpu.VMEM((2, 2 * JR, IW), jnp.int32),
        pltpu.VMEM((2, PAIR), jnp.float32),
        pltpu.VMEM((2, CH, H), jnp.float32),
        pltpu.VMEM((ZR, H), jnp.float32),
        pltpu.SemaphoreType.DMA,
        pltpu.SemaphoreType.DMA,
        pltpu.SemaphoreType.DMA,
        pltpu.SemaphoreType.DMA,
    ],
)
def _gcn_sc(xs, src2, dst2, wf, out, acc_sp, src_vm, dst_vm, w_vm, rows_vm,
            zeros_vm, gsem0, gsem1, ssem0, ssem1):
    c = lax.axis_index("c")
    s = lax.axis_index("s")
    row0 = s * RPT
    gsems = (gsem0, gsem1)
    ssems = (ssem0, ssem1)

    zv = jnp.zeros((16,), jnp.float32)

    def zbody(r, carry):
        for k in range(H // 16):
            zeros_vm[r, pl.ds(k * 16, 16)] = zv
        return carry

    lax.fori_loop(0, ZR, zbody, 0)

    def zero_acc():
        for z in range(RPT // ZR):
            pltpu.sync_copy(zeros_vm, acc_sp.at[pl.ds(row0 + z * ZR, ZR)])

    zero_acc()
    plsc.subcore_barrier()

    def do_layer(h_ref, lyr):
        dummy = out.at[0].at[pl.ds(0, CH)]

        def drain_g(b):
            pltpu.make_async_copy(dummy, rows_vm.at[b], gsems[b]).wait()

        def drain_s(b):
            pltpu.make_async_copy(dummy, rows_vm.at[b], ssems[b]).wait()

        def load_idx(p, pu):
            i0 = s * (EPT // IW) + p * (2 * JR)
            e0 = s * EPT + p * PAIR
            pltpu.sync_copy(src2.at[c].at[pl.ds(i0, 2 * JR)], src_vm.at[pu])
            pltpu.sync_copy(dst2.at[pl.ds(i0, 2 * JR)], dst_vm.at[pu])
            pltpu.sync_copy(wf.at[pl.ds(e0, PAIR)], w_vm.at[pu])

        def fire_g(b, pu):
            for j in range(JR):
                pltpu.async_copy(h_ref.at[src_vm.at[pu].at[b * JR + j]],
                                 rows_vm.at[b].at[pl.ds(j * IW, IW)],
                                 gsems[b])

        def fire_s(b, pu):
            for j in range(JR):
                pltpu.async_copy(rows_vm.at[b].at[pl.ds(j * IW, IW)],
                                 acc_sp.at[dst_vm.at[pu].at[b * JR + j]],
                                 ssems[b], add=True)

        def scale(b, pu):
            def s16(g, carry2):
                wv = w_vm[pu, pl.ds(b * CH + g * 16, 16)]
                for j in range(16):
                    w = wv[j]
                    e = g * 16 + j
                    for k in range(H // 16):
                        sl = pl.ds(k * 16, 16)
                        rows_vm[b, e, sl] = rows_vm[b, e, sl] * w
                return carry2

            lax.fori_loop(0, CH // 16, s16, 0)


        load_idx(0, 0)
        fire_g(0, 0)
        drain_g(0)
        fire_g(1, 0)
        scale(0, 0)
        fire_s(0, 0)
        drain_g(1)
        drain_s(0)
        load_idx(1, 1)
        fire_g(0, 1)
        scale(1, 0)
        fire_s(1, 0)

        def super_body(t, carry):
            drain_g(0)
            drain_s(1)
            fire_g(1, 0)
            scale(0, 0)
            fire_s(0, 0)
            drain_g(1)
            drain_s(0)
            load_idx(2 * t + 1, 1)
            fire_g(0, 1)
            scale(1, 0)
            fire_s(1, 0)
            drain_g(0)
            drain_s(1)
            fire_g(1, 1)
            scale(0, 1)
            fire_s(0, 1)
            drain_g(1)
            drain_s(0)
            load_idx(2 * t + 2, 0)
            fire_g(0, 0)
            scale(1, 1)
            fire_s(1, 1)
            return carry

        drain_g(0)
        drain_s(1)
        fire_g(1, 1)
        scale(0, 1)
        fire_s(0, 1)
        drain_g(1)
        drain_s(0)
        load_idx(2, 0)
        fire_g(0, 0)
        scale(1, 1)
        fire_s(1, 1)

        lax.fori_loop(1, NPAIR // 2, super_body, 0)

        drain_g(0)
        drain_s(1)
        fire_g(1, 0)
        scale(0, 0)
        fire_s(0, 0)
        drain_g(1)
        drain_s(0)
        scale(1, 0)
        fire_s(1, 0)
        drain_s(1)
        plsc.subcore_barrier()

        pltpu.sync_copy(acc_sp.at[pl.ds(row0, RPT)],
                        out.at[lyr].at[pl.ds(c * NP + row0, RPT)])
        if lyr < L - 1:
            zero_acc()
        plsc.subcore_barrier()

    do_layer(xs, 0)
    do_layer(out.at[0], 1)
    do_layer(out.at[1], 2)


RB = 640


def _mm_body(hs_ref, wt_ref, b_ref, o_ref):
    hs = hs_ref[...]
    s0 = hs[0, 0] + hs[1, 0] + hs[2, 0]
    s1 = hs[0, 1] + hs[1, 1] + hs[2, 1]
    m = jnp.concatenate([s0, s1], axis=-1) * (1.0 / 3.0)
    o_ref[...] = jnp.dot(m, wt_ref[...],
                         preferred_element_type=jnp.float32) + b_ref[...]


def _mean_linear(hs4, Wt, b2):
    return pl.pallas_call(
        _mm_body,
        grid=(NP // RB,),
        in_specs=[
            pl.BlockSpec((L, NC, RB, H), lambda i: (0, 0, i, 0)),
            pl.BlockSpec((F, F), lambda i: (0, 0)),
            pl.BlockSpec((1, F), lambda i: (0, 0)),
        ],
        out_specs=pl.BlockSpec((RB, F), lambda i: (i, 0)),
        out_shape=jax.ShapeDtypeStruct((NP, F), jnp.float32),
    )(hs4, Wt, b2)


@jax.jit
def kernel(x, edge_index, edge_weight, W, b):
    xs = x.reshape(N, NC, H).transpose(1, 0, 2)
    xs = jnp.pad(xs, ((0, 0), (0, NP - N), (0, 0))).reshape(NC * NP, H)
    src = edge_index[1]
    src2 = (src[None, :] + jnp.array([[0], [NP]], jnp.int32)).reshape(
        NC, E // IW, IW)
    dst2 = edge_index[0].reshape(E // IW, IW)
    hs = _gcn_sc(xs, src2, dst2, edge_weight)
    hs4 = hs.reshape(L, NC, NP, H)
    return _mean_linear(hs4, W.T, b.reshape(1, F))[:N]
